# baseline (speedup 1.0000x reference)
import functools

import jax
import jax.numpy as jnp
from jax import lax
from jax.experimental import pallas as pl
from jax.experimental.pallas import tpu as pltpu
from jax.experimental.pallas import tpu_sc as plsc

N_NODES_ = 10000
D_ = 256
E_ = 160000
INV_T = 10.0

NC_ = 2
NS_ = 16
NW_ = NC_ * NS_
K_ = 128
CW_ = 40
EW_ = K_ * CW_
E_PAD_ = NW_ * EW_
DW_ = D_ // 2


def _norm_body(z_ref, zn_ref):
    x = z_ref[...]
    n2 = jnp.sum(x * x, axis=1, keepdims=True)
    inv = lax.rsqrt(jnp.maximum(n2, 1e-16))
    zn_ref[...] = (x * inv).astype(jnp.bfloat16)


def _normalize(z):
    return pl.pallas_call(
        _norm_body,
        out_shape=jax.ShapeDtypeStruct((N_NODES_, D_), jnp.bfloat16),
    )(z)


def _sc_body(zi, pa, pb, na, nb, ps, ns, idxa, idxb, ra, rb, sims, sem):
    wid = lax.axis_index("s") * NC_ + lax.axis_index("c")
    iota16 = lax.iota(jnp.int32, 16)

    def run_sign(ia_hbm, ib_hbm, out_hbm):
        base = pl.multiple_of(wid * EW_, EW_)
        pltpu.sync_copy(ia_hbm.at[pl.ds(base, EW_)], idxa)
        pltpu.sync_copy(ib_hbm.at[pl.ds(base, EW_)], idxb)

        def chunk(ci, carry):
            lbase = pl.multiple_of(ci * K_, K_)
            ca = pltpu.async_copy(zi.at[idxa.at[pl.ds(lbase, K_)]], ra, sem)
            cb = pltpu.async_copy(zi.at[idxb.at[pl.ds(lbase, K_)]], rb, sem)
            ca.wait()
            cb.wait()
            for g in range(K_ // 16):
                rows16 = g * 16 + iota16

                def jbody(j, accs):
                    acc0, acc1 = accs
                    col = jnp.full((16,), 0, jnp.int32) + j
                    wa = plsc.load_gather(ra, [rows16, col])
                    wb = plsc.load_gather(rb, [rows16, col])
                    a0, a1 = plsc.unpack(
                        plsc.bitcast(wa, jnp.bfloat16),
                        format=plsc.PackFormat.INTERLEAVED,
                    )
                    b0, b1 = plsc.unpack(
                        plsc.bitcast(wb, jnp.bfloat16),
                        format=plsc.PackFormat.INTERLEAVED,
                    )
                    return acc0 + a0 * b0, acc1 + a1 * b1

                zero16 = jnp.zeros((16,), jnp.float32)
                acc0, acc1 = lax.fori_loop(
                    0, DW_, jbody, (zero16, zero16), unroll=8
                )
                sims[pl.ds(lbase + g * 16, 16)] = (acc0 + acc1) * INV_T
            return carry

        lax.fori_loop(0, CW_, chunk, 0)
        pltpu.sync_copy(sims, out_hbm.at[pl.ds(base, EW_)])

    run_sign(pa, pb, ps)
    run_sign(na, nb, ns)


def _sc_sims(zi, pa, pb, na, nb):
    return pl.kernel(
        _sc_body,
        out_type=[
            jax.ShapeDtypeStruct((E_PAD_,), jnp.float32),
            jax.ShapeDtypeStruct((E_PAD_,), jnp.float32),
        ],
        mesh=plsc.VectorSubcoreMesh(
            core_axis_name="c", subcore_axis_name="s"
        ),
        compiler_params=pltpu.CompilerParams(
            needs_layout_passes=False, use_tc_tiling_on_sc=False
        ),
        scratch_types=[
            pltpu.VMEM((EW_,), jnp.int32),
            pltpu.VMEM((EW_,), jnp.int32),
            pltpu.VMEM((K_, DW_), jnp.int32),
            pltpu.VMEM((K_, DW_), jnp.int32),
            pltpu.VMEM((EW_,), jnp.float32),
            pltpu.SemaphoreType.DMA,
        ],
    )(zi, pa, pb, na, nb)


def _loss_body(ps_ref, ns_ref, out_ref, *, n_valid):
    rows, cols = ps_ref.shape
    ridx = lax.broadcasted_iota(jnp.int32, (rows, cols), 0)
    cidx = lax.broadcasted_iota(jnp.int32, (rows, cols), 1)
    valid = (ridx * cols + cidx) < n_valid
    ns = ns_ref[...]
    nsum = jnp.sum(jnp.where(valid, jnp.exp(ns), 0.0))
    ps = ps_ref[...]
    loss = jnp.where(valid, jnp.log(jnp.exp(ps) + nsum) - ps, 0.0)
    out_ref[...] = (jnp.sum(loss) / n_valid).reshape(1, 1)


def _loss(ps2d, ns2d, n_valid):
    out = pl.pallas_call(
        functools.partial(_loss_body, n_valid=n_valid),
        out_shape=jax.ShapeDtypeStruct((1, 1), jnp.float32),
    )(ps2d, ns2d)
    return out.reshape(())


def _pad_idx(v):
    return jnp.concatenate(
        [v, jnp.full((E_PAD_ - E_,), N_NODES_, jnp.int32)]
    )


def kernel(z, edge_index, negative_edge_index):
    zn = _normalize(z)
    zn_pad = jnp.concatenate(
        [zn, jnp.zeros((1, D_), jnp.bfloat16)], axis=0
    )
    zi = lax.bitcast_convert_type(
        zn_pad.reshape(N_NODES_ + 1, DW_, 2), jnp.int32
    )
    pa = _pad_idx(edge_index[0])
    pb = _pad_idx(edge_index[1])
    na = _pad_idx(negative_edge_index[0])
    nb = _pad_idx(negative_edge_index[1])
    ps, ns = _sc_sims(zi, pa, pb, na, nb)
    return _loss(
        ps.reshape(E_PAD_ // 128, 128), ns.reshape(E_PAD_ // 128, 128), E_
    )


# speedup vs baseline: 2.2058x; 2.2058x over previous
import functools

import jax
import jax.numpy as jnp
from jax import lax
from jax.experimental import pallas as pl
from jax.experimental.pallas import tpu as pltpu
from jax.experimental.pallas import tpu_sc as plsc

N_NODES_ = 10000
D_ = 256
E_ = 160000
INV_T = 10.0

NC_ = 2
NS_ = 16
NW_ = NC_ * NS_
K_ = 128
CW_ = 40
EW_ = K_ * CW_
E_PAD_ = NW_ * EW_
DW_ = D_ // 2


def _norm_body(z_ref, zn_ref):
    x = z_ref[...]
    n2 = jnp.sum(x * x, axis=1, keepdims=True)
    inv = lax.rsqrt(jnp.maximum(n2, 1e-16))
    zn_ref[...] = (x * inv).astype(jnp.bfloat16)


def _normalize(z):
    return pl.pallas_call(
        _norm_body,
        out_shape=jax.ShapeDtypeStruct((N_NODES_, D_), jnp.bfloat16),
    )(z)


def _sc_body(
    zi, pa, pb, na, nb, ps, ns, idxa, idxb, ra0, rb0, ra1, rb1, sims, sem0, sem1
):
    wid = lax.axis_index("s") * NC_ + lax.axis_index("c")
    iota16 = lax.iota(jnp.int32, 16)
    slots = ((ra0, rb0, sem0), (ra1, rb1, sem1))

    def run_sign(ia_hbm, ib_hbm, out_hbm):
        base = pl.multiple_of(wid * EW_, EW_)
        pltpu.sync_copy(ia_hbm.at[pl.ds(base, EW_)], idxa)
        pltpu.sync_copy(ib_hbm.at[pl.ds(base, EW_)], idxb)

        def start(ci, slot):
            ra, rb, sem = slot
            lb = pl.multiple_of(ci * K_, K_)
            pltpu.async_copy(zi.at[idxa.at[pl.ds(lb, K_)]], ra, sem)
            pltpu.async_copy(zi.at[idxb.at[pl.ds(lb, K_)]], rb, sem)

        def wait(slot):
            ra, rb, sem = slot
            pltpu.make_async_copy(zi.at[pl.ds(0, K_)], ra, sem).wait()
            pltpu.make_async_copy(zi.at[pl.ds(0, K_)], rb, sem).wait()

        def compute(ci, slot):
            ra, rb, _ = slot
            lb = pl.multiple_of(ci * K_, K_)

            def gbody(g, carry):
                sims16 = jnp.zeros((16,), jnp.float32)
                for i in range(16):
                    e = g * 16 + i
                    acc0 = acc1 = None
                    for t in range(8):
                        a = plsc.bitcast(
                            ra[e, pl.ds(t * 16, 16)], jnp.bfloat16
                        )
                        b = plsc.bitcast(
                            rb[e, pl.ds(t * 16, 16)], jnp.bfloat16
                        )
                        p = a * b
                        if t % 2 == 0:
                            acc0 = p if acc0 is None else acc0 + p
                        else:
                            acc1 = p if acc1 is None else acc1 + p
                    lo, hi = plsc.unpack(
                        acc0 + acc1, format=plsc.PackFormat.INTERLEAVED
                    )
                    d = jnp.sum(lo + hi)
                    sims16 = jnp.where(iota16 == i, d, sims16)
                off = pl.multiple_of(lb + g * 16, 16)
                sims[pl.ds(off, 16)] = sims16 * INV_T
                return carry

            lax.fori_loop(0, K_ // 16, gbody, 0)

        start(0, slots[0])

        def pair(p2, carry):
            for b2 in range(2):
                ci = p2 * 2 + b2
                nxt = ci + 1

                @pl.when(nxt < CW_)
                def _():
                    start(nxt, slots[1 - b2])

                wait(slots[b2])
                compute(ci, slots[b2])
            return carry

        lax.fori_loop(0, CW_ // 2, pair, 0)
        pltpu.sync_copy(sims, out_hbm.at[pl.ds(base, EW_)])

    run_sign(pa, pb, ps)
    run_sign(na, nb, ns)


def _sc_sims(zi, pa, pb, na, nb):
    return pl.kernel(
        _sc_body,
        out_type=[
            jax.ShapeDtypeStruct((E_PAD_,), jnp.float32),
            jax.ShapeDtypeStruct((E_PAD_,), jnp.float32),
        ],
        mesh=plsc.VectorSubcoreMesh(
            core_axis_name="c", subcore_axis_name="s"
        ),
        compiler_params=pltpu.CompilerParams(
            needs_layout_passes=False, use_tc_tiling_on_sc=False
        ),
        scratch_types=[
            pltpu.VMEM((EW_,), jnp.int32),
            pltpu.VMEM((EW_,), jnp.int32),
            pltpu.VMEM((K_, DW_), jnp.int32),
            pltpu.VMEM((K_, DW_), jnp.int32),
            pltpu.VMEM((K_, DW_), jnp.int32),
            pltpu.VMEM((K_, DW_), jnp.int32),
            pltpu.VMEM((EW_,), jnp.float32),
            pltpu.SemaphoreType.DMA,
            pltpu.SemaphoreType.DMA,
        ],
    )(zi, pa, pb, na, nb)


def _loss_body(ps_ref, ns_ref, out_ref, *, n_valid):
    rows, cols = ps_ref.shape
    ridx = lax.broadcasted_iota(jnp.int32, (rows, cols), 0)
    cidx = lax.broadcasted_iota(jnp.int32, (rows, cols), 1)
    valid = (ridx * cols + cidx) < n_valid
    ns = ns_ref[...]
    nsum = jnp.sum(jnp.where(valid, jnp.exp(ns), 0.0))
    ps = ps_ref[...]
    loss = jnp.where(valid, jnp.log(jnp.exp(ps) + nsum) - ps, 0.0)
    out_ref[...] = (jnp.sum(loss) / n_valid).reshape(1, 1)


def _loss(ps2d, ns2d, n_valid):
    out = pl.pallas_call(
        functools.partial(_loss_body, n_valid=n_valid),
        out_shape=jax.ShapeDtypeStruct((1, 1), jnp.float32),
    )(ps2d, ns2d)
    return out.reshape(())


def _pad_idx(v):
    return jnp.concatenate(
        [v, jnp.full((E_PAD_ - E_,), N_NODES_, jnp.int32)]
    )


def kernel(z, edge_index, negative_edge_index):
    zn = _normalize(z)
    zn_pad = jnp.concatenate(
        [zn, jnp.zeros((1, D_), jnp.bfloat16)], axis=0
    )
    zi = lax.bitcast_convert_type(
        zn_pad.reshape(N_NODES_ + 1, DW_, 2), jnp.int32
    )
    pa = _pad_idx(edge_index[0])
    pb = _pad_idx(edge_index[1])
    na = _pad_idx(negative_edge_index[0])
    nb = _pad_idx(negative_edge_index[1])
    ps, ns = _sc_sims(zi, pa, pb, na, nb)
    return _loss(
        ps.reshape(E_PAD_ // 128, 128), ns.reshape(E_PAD_ // 128, 128), E_
    )


# speedup vs baseline: 3.7763x; 1.7120x over previous
import functools

import jax
import jax.numpy as jnp
from jax import lax
from jax.experimental import pallas as pl
from jax.experimental.pallas import tpu as pltpu
from jax.experimental.pallas import tpu_sc as plsc

N_NODES_ = 10000
D_ = 256
E_ = 160000
INV_T = 10.0

NC_ = 2
NS_ = 16
NW_ = NC_ * NS_
K_ = 128
CW_ = 40
EW_ = K_ * CW_
E_PAD_ = NW_ * EW_
DW_ = D_ // 4


def _norm_body(z_ref, zn_ref):
    x = z_ref[...]
    n2 = jnp.sum(x * x, axis=1, keepdims=True)
    inv = lax.rsqrt(jnp.maximum(n2, 1e-16))
    zn_ref[...] = (x * inv).astype(jnp.float8_e4m3fn)


def _normalize(z):
    return pl.pallas_call(
        _norm_body,
        out_shape=jax.ShapeDtypeStruct((N_NODES_, D_), jnp.float8_e4m3fn),
    )(z)


def _sc_body(
    zi, pa, pb, na, nb, ps, ns, idxa, idxb, ra0, rb0, ra1, rb1, sims, sem0, sem1
):
    wid = lax.axis_index("s") * NC_ + lax.axis_index("c")
    iota16 = lax.iota(jnp.int32, 16)
    slots = ((ra0, rb0, sem0), (ra1, rb1, sem1))

    def run_sign(ia_hbm, ib_hbm, out_hbm):
        base = pl.multiple_of(wid * EW_, EW_)
        pltpu.sync_copy(ia_hbm.at[pl.ds(base, EW_)], idxa)
        pltpu.sync_copy(ib_hbm.at[pl.ds(base, EW_)], idxb)

        def start(ci, slot):
            ra, rb, sem = slot
            lb = pl.multiple_of(ci * K_, K_)
            pltpu.async_copy(zi.at[idxa.at[pl.ds(lb, K_)]], ra, sem)
            pltpu.async_copy(zi.at[idxb.at[pl.ds(lb, K_)]], rb, sem)

        def wait(slot):
            ra, rb, sem = slot
            pltpu.make_async_copy(zi.at[pl.ds(0, K_)], ra, sem).wait()
            pltpu.make_async_copy(zi.at[pl.ds(0, K_)], rb, sem).wait()

        def compute(ci, slot):
            ra, rb, _ = slot
            lb = pl.multiple_of(ci * K_, K_)

            def gbody(g, carry):
                sims16 = jnp.zeros((16,), jnp.float32)
                for i in range(16):
                    e = g * 16 + i
                    acc0 = acc1 = None
                    for t in range(4):
                        a8 = plsc.bitcast(
                            ra[e, pl.ds(t * 16, 16)], jnp.float8_e4m3fn
                        )
                        b8 = plsc.bitcast(
                            rb[e, pl.ds(t * 16, 16)], jnp.float8_e4m3fn
                        )
                        alo, ahi = plsc.unpack(
                            a8,
                            format=plsc.PackFormat.INTERLEAVED,
                            preferred_element_type=jnp.bfloat16,
                        )
                        blo, bhi = plsc.unpack(
                            b8,
                            format=plsc.PackFormat.INTERLEAVED,
                            preferred_element_type=jnp.bfloat16,
                        )
                        p0 = alo * blo
                        p1 = ahi * bhi
                        acc0 = p0 if acc0 is None else acc0 + p0
                        acc1 = p1 if acc1 is None else acc1 + p1
                    lo, hi = plsc.unpack(
                        acc0 + acc1, format=plsc.PackFormat.INTERLEAVED
                    )
                    d = jnp.sum(lo + hi)
                    sims16 = jnp.where(iota16 == i, d, sims16)
                off = pl.multiple_of(lb + g * 16, 16)
                sims[pl.ds(off, 16)] = sims16 * INV_T
                return carry

            lax.fori_loop(0, K_ // 16, gbody, 0)

        start(0, slots[0])

        def pair(p2, carry):
            for b2 in range(2):
                ci = p2 * 2 + b2
                nxt = ci + 1

                @pl.when(nxt < CW_)
                def _():
                    start(nxt, slots[1 - b2])

                wait(slots[b2])
                compute(ci, slots[b2])
            return carry

        lax.fori_loop(0, CW_ // 2, pair, 0)
        pltpu.sync_copy(sims, out_hbm.at[pl.ds(base, EW_)])

    run_sign(pa, pb, ps)
    run_sign(na, nb, ns)


def _sc_sims(zi, pa, pb, na, nb):
    return pl.kernel(
        _sc_body,
        out_type=[
            jax.ShapeDtypeStruct((E_PAD_,), jnp.float32),
            jax.ShapeDtypeStruct((E_PAD_,), jnp.float32),
        ],
        mesh=plsc.VectorSubcoreMesh(
            core_axis_name="c", subcore_axis_name="s"
        ),
        compiler_params=pltpu.CompilerParams(
            needs_layout_passes=False, use_tc_tiling_on_sc=False
        ),
        scratch_types=[
            pltpu.VMEM((EW_,), jnp.int32),
            pltpu.VMEM((EW_,), jnp.int32),
            pltpu.VMEM((K_, DW_), jnp.int32),
            pltpu.VMEM((K_, DW_), jnp.int32),
            pltpu.VMEM((K_, DW_), jnp.int32),
            pltpu.VMEM((K_, DW_), jnp.int32),
            pltpu.VMEM((EW_,), jnp.float32),
            pltpu.SemaphoreType.DMA,
            pltpu.SemaphoreType.DMA,
        ],
    )(zi, pa, pb, na, nb)


def _loss_body(ps_ref, ns_ref, out_ref, *, n_valid):
    rows, cols = ps_ref.shape
    ridx = lax.broadcasted_iota(jnp.int32, (rows, cols), 0)
    cidx = lax.broadcasted_iota(jnp.int32, (rows, cols), 1)
    valid = (ridx * cols + cidx) < n_valid
    ns = ns_ref[...]
    nsum = jnp.sum(jnp.where(valid, jnp.exp(ns), 0.0))
    ps = ps_ref[...]
    loss = jnp.where(valid, jnp.log(jnp.exp(ps) + nsum) - ps, 0.0)
    out_ref[...] = (jnp.sum(loss) / n_valid).reshape(1, 1)


def _loss(ps2d, ns2d, n_valid):
    out = pl.pallas_call(
        functools.partial(_loss_body, n_valid=n_valid),
        out_shape=jax.ShapeDtypeStruct((1, 1), jnp.float32),
    )(ps2d, ns2d)
    return out.reshape(())


def _pad_idx(v):
    return jnp.concatenate(
        [v, jnp.full((E_PAD_ - E_,), N_NODES_, jnp.int32)]
    )


def kernel(z, edge_index, negative_edge_index):
    zn = _normalize(z)
    zn_pad = jnp.concatenate(
        [zn, jnp.zeros((1, D_), jnp.float8_e4m3fn)], axis=0
    )
    zi = lax.bitcast_convert_type(
        zn_pad.reshape(N_NODES_ + 1, DW_, 4), jnp.int32
    )
    pa = _pad_idx(edge_index[0])
    pb = _pad_idx(edge_index[1])
    na = _pad_idx(negative_edge_index[0])
    nb = _pad_idx(negative_edge_index[1])
    ps, ns = _sc_sims(zi, pa, pb, na, nb)
    return _loss(
        ps.reshape(E_PAD_ // 128, 128), ns.reshape(E_PAD_ // 128, 128), E_
    )


# speedup vs baseline: 12.2045x; 3.2319x over previous
import functools

import jax
import jax.numpy as jnp
from jax import lax
from jax.experimental import pallas as pl
from jax.experimental.pallas import tpu as pltpu
from jax.experimental.pallas import tpu_sc as plsc

N_NODES_ = 10000
D_ = 256
E_ = 160000
INV_T = 10.0

NC_ = 2
NS_ = 16
NW_ = NC_ * NS_
K_ = 128
CW_ = 40
EW_ = K_ * CW_
E_PAD_ = NW_ * EW_
DW_ = D_ // 4
NROWS_ = 10016


def _norm_body(z_ref, zn_ref):
    x = z_ref[...]
    n2 = jnp.sum(x * x, axis=1, keepdims=True)
    inv = lax.rsqrt(jnp.maximum(n2, 1e-16))
    zn_ref[...] = (x * inv).astype(jnp.float8_e4m3fn)


def _normalize(z):
    return pl.pallas_call(
        _norm_body,
        out_shape=jax.ShapeDtypeStruct((N_NODES_, D_), jnp.float8_e4m3fn),
    )(z)


def _sc_body(
    zi, pa, pb, na, nb, ps, ns,
    idxa, idxb, ra0, rb0, ra1, rb1, sims, zs, sem0, sem1,
):
    sid = lax.axis_index("s")
    wid = sid * NC_ + lax.axis_index("c")
    iota16 = lax.iota(jnp.int32, 16)
    slots = ((ra0, rb0, sem0), (ra1, rb1, sem1))

    rows_per_tile = NROWS_ // NS_
    roff = pl.multiple_of(sid * rows_per_tile, rows_per_tile)
    pltpu.sync_copy(zi.at[pl.ds(roff, rows_per_tile)],
                    zs.at[pl.ds(roff, rows_per_tile)])
    plsc.subcore_barrier()

    def run_sign(ia_hbm, ib_hbm, out_hbm):
        base = pl.multiple_of(wid * EW_, EW_)
        pltpu.sync_copy(ia_hbm.at[pl.ds(base, EW_)], idxa)
        pltpu.sync_copy(ib_hbm.at[pl.ds(base, EW_)], idxb)

        def start(ci, slot):
            ra, rb, sem = slot
            lb = pl.multiple_of(ci * K_, K_)
            pltpu.async_copy(zs.at[idxa.at[pl.ds(lb, K_)]], ra, sem)
            pltpu.async_copy(zs.at[idxb.at[pl.ds(lb, K_)]], rb, sem)

        def wait(slot):
            ra, rb, sem = slot
            pltpu.make_async_copy(zi.at[pl.ds(0, K_)], ra, sem).wait()
            pltpu.make_async_copy(zi.at[pl.ds(0, K_)], rb, sem).wait()

        def compute(ci, slot):
            ra, rb, _ = slot
            lb = pl.multiple_of(ci * K_, K_)

            def gbody(g, carry):
                sims16 = jnp.zeros((16,), jnp.float32)
                for i in range(16):
                    e = g * 16 + i
                    acc0 = acc1 = None
                    for t in range(4):
                        a8 = plsc.bitcast(
                            ra[e, pl.ds(t * 16, 16)], jnp.float8_e4m3fn
                        )
                        b8 = plsc.bitcast(
                            rb[e, pl.ds(t * 16, 16)], jnp.float8_e4m3fn
                        )
                        alo, ahi = plsc.unpack(
                            a8,
                            format=plsc.PackFormat.INTERLEAVED,
                            preferred_element_type=jnp.bfloat16,
                        )
                        blo, bhi = plsc.unpack(
                            b8,
                            format=plsc.PackFormat.INTERLEAVED,
                            preferred_element_type=jnp.bfloat16,
                        )
                        p0 = alo * blo
                        p1 = ahi * bhi
                        acc0 = p0 if acc0 is None else acc0 + p0
                        acc1 = p1 if acc1 is None else acc1 + p1
                    lo, hi = plsc.unpack(
                        acc0 + acc1, format=plsc.PackFormat.INTERLEAVED
                    )
                    d = jnp.sum(lo + hi)
                    sims16 = jnp.where(iota16 == i, d, sims16)
                off = pl.multiple_of(lb + g * 16, 16)
                sims[pl.ds(off, 16)] = sims16 * INV_T
                return carry

            lax.fori_loop(0, K_ // 16, gbody, 0)

        start(0, slots[0])

        def pair(p2, carry):
            for b2 in range(2):
                ci = p2 * 2 + b2
                nxt = ci + 1

                @pl.when(nxt < CW_)
                def _():
                    start(nxt, slots[1 - b2])

                wait(slots[b2])
                compute(ci, slots[b2])
            return carry

        lax.fori_loop(0, CW_ // 2, pair, 0)
        pltpu.sync_copy(sims, out_hbm.at[pl.ds(base, EW_)])

    run_sign(pa, pb, ps)
    run_sign(na, nb, ns)


def _sc_sims(zi, pa, pb, na, nb):
    return pl.kernel(
        _sc_body,
        out_type=[
            jax.ShapeDtypeStruct((E_PAD_,), jnp.float32),
            jax.ShapeDtypeStruct((E_PAD_,), jnp.float32),
        ],
        mesh=plsc.VectorSubcoreMesh(
            core_axis_name="c", subcore_axis_name="s"
        ),
        compiler_params=pltpu.CompilerParams(
            needs_layout_passes=False, use_tc_tiling_on_sc=False
        ),
        scratch_types=[
            pltpu.VMEM((EW_,), jnp.int32),
            pltpu.VMEM((EW_,), jnp.int32),
            pltpu.VMEM((K_, DW_), jnp.int32),
            pltpu.VMEM((K_, DW_), jnp.int32),
            pltpu.VMEM((K_, DW_), jnp.int32),
            pltpu.VMEM((K_, DW_), jnp.int32),
            pltpu.VMEM((EW_,), jnp.float32),
            pltpu.VMEM_SHARED((NROWS_, DW_), jnp.int32),
            pltpu.SemaphoreType.DMA,
            pltpu.SemaphoreType.DMA,
        ],
    )(zi, pa, pb, na, nb)


def _loss_body(ps_ref, ns_ref, out_ref, *, n_valid):
    rows, cols = ps_ref.shape
    ridx = lax.broadcasted_iota(jnp.int32, (rows, cols), 0)
    cidx = lax.broadcasted_iota(jnp.int32, (rows, cols), 1)
    valid = (ridx * cols + cidx) < n_valid
    ns = ns_ref[...]
    nsum = jnp.sum(jnp.where(valid, jnp.exp(ns), 0.0))
    ps = ps_ref[...]
    loss = jnp.where(valid, jnp.log(jnp.exp(ps) + nsum) - ps, 0.0)
    out_ref[...] = (jnp.sum(loss) / n_valid).reshape(1, 1)


def _loss(ps2d, ns2d, n_valid):
    out = pl.pallas_call(
        functools.partial(_loss_body, n_valid=n_valid),
        out_shape=jax.ShapeDtypeStruct((1, 1), jnp.float32),
    )(ps2d, ns2d)
    return out.reshape(())


def _pad_idx(v):
    return jnp.concatenate(
        [v, jnp.full((E_PAD_ - E_,), N_NODES_, jnp.int32)]
    )


def kernel(z, edge_index, negative_edge_index):
    zn = _normalize(z)
    zn_pad = jnp.concatenate(
        [zn, jnp.zeros((NROWS_ - N_NODES_, D_), jnp.float8_e4m3fn)], axis=0
    )
    zi = lax.bitcast_convert_type(
        zn_pad.reshape(NROWS_, DW_, 4), jnp.int32
    )
    pa = _pad_idx(edge_index[0])
    pb = _pad_idx(edge_index[1])
    na = _pad_idx(negative_edge_index[0])
    nb = _pad_idx(negative_edge_index[1])
    ps, ns = _sc_sims(zi, pa, pb, na, nb)
    return _loss(
        ps.reshape(E_PAD_ // 128, 128), ns.reshape(E_PAD_ // 128, 128), E_
    )


# speedup vs baseline: 15.5696x; 1.2757x over previous
import functools

import jax
import jax.numpy as jnp
from jax import lax
from jax.experimental import pallas as pl
from jax.experimental.pallas import tpu as pltpu
from jax.experimental.pallas import tpu_sc as plsc

N_NODES_ = 10000
D_ = 256
E_ = 160000
INV_T = 10.0

NC_ = 2
NS_ = 16
NW_ = NC_ * NS_
K_ = 128
CW_ = 40
EW_ = K_ * CW_
E_PAD_ = NW_ * EW_
NROWS_ = 10016


def _norm_body(z_ref, zn_ref):
    x = z_ref[...]
    n2 = jnp.sum(x * x, axis=1, keepdims=True)
    inv = lax.rsqrt(jnp.maximum(n2, 1e-16))
    zn_ref[...] = (x * inv).astype(jnp.float8_e4m3fn)


def _normalize(z):
    return pl.pallas_call(
        _norm_body,
        out_shape=jax.ShapeDtypeStruct((N_NODES_, D_), jnp.float8_e4m3fn),
    )(z)


def _sc_body(
    zi, pa, pb, na, nb, ps, ns,
    idxa, idxb, ra0, rb0, ra1, rb1, sims, zs, sem0, sem1,
):
    sid = lax.axis_index("s")
    wid = sid * NC_ + lax.axis_index("c")
    iota16 = lax.iota(jnp.int32, 16)
    slots = ((ra0, rb0, sem0), (ra1, rb1, sem1))

    rows_per_tile = NROWS_ // NS_
    roff = pl.multiple_of(sid * rows_per_tile, rows_per_tile)
    pltpu.sync_copy(zi.at[pl.ds(roff, rows_per_tile)],
                    zs.at[pl.ds(roff, rows_per_tile)])
    plsc.subcore_barrier()

    def run_sign(ia_hbm, ib_hbm, out_hbm):
        base = pl.multiple_of(wid * EW_, EW_)
        pltpu.sync_copy(ia_hbm.at[pl.ds(base, EW_)], idxa)
        pltpu.sync_copy(ib_hbm.at[pl.ds(base, EW_)], idxb)

        def start(ci, slot):
            ra, rb, sem = slot
            lb = pl.multiple_of(ci * K_, K_)
            pltpu.async_copy(zs.at[idxa.at[pl.ds(lb, K_)]], ra, sem)
            pltpu.async_copy(zs.at[idxb.at[pl.ds(lb, K_)]], rb, sem)

        def wait(slot):
            ra, rb, sem = slot
            pltpu.make_async_copy(zi.at[pl.ds(0, K_)], ra, sem).wait()
            pltpu.make_async_copy(zi.at[pl.ds(0, K_)], rb, sem).wait()

        def compute(ci, slot):
            ra, rb, _ = slot
            lb = pl.multiple_of(ci * K_, K_)

            def gbody(g, carry):
                sims16 = jnp.zeros((16,), jnp.float32)
                for i in range(16):
                    e = g * 16 + i
                    acc0 = acc1 = None
                    for t in range(4):
                        a8 = ra[e, pl.ds(t * 64, 64)]
                        b8 = rb[e, pl.ds(t * 64, 64)]
                        alo, ahi = plsc.unpack(
                            a8,
                            format=plsc.PackFormat.INTERLEAVED,
                            preferred_element_type=jnp.bfloat16,
                        )
                        blo, bhi = plsc.unpack(
                            b8,
                            format=plsc.PackFormat.INTERLEAVED,
                            preferred_element_type=jnp.bfloat16,
                        )
                        p0 = alo * blo
                        p1 = ahi * bhi
                        acc0 = p0 if acc0 is None else acc0 + p0
                        acc1 = p1 if acc1 is None else acc1 + p1
                    lo, hi = plsc.unpack(
                        acc0 + acc1, format=plsc.PackFormat.INTERLEAVED
                    )
                    d = jnp.sum(lo + hi)
                    sims16 = jnp.where(iota16 == i, d, sims16)
                off = pl.multiple_of(lb + g * 16, 16)
                sims[pl.ds(off, 16)] = sims16 * INV_T
                return carry

            lax.fori_loop(0, K_ // 16, gbody, 0)

        start(0, slots[0])

        def pair(p2, carry):
            for b2 in range(2):
                ci = p2 * 2 + b2
                nxt = ci + 1

                @pl.when(nxt < CW_)
                def _():
                    start(nxt, slots[1 - b2])

                wait(slots[b2])
                compute(ci, slots[b2])
            return carry

        lax.fori_loop(0, CW_ // 2, pair, 0)
        pltpu.sync_copy(sims, out_hbm.at[pl.ds(base, EW_)])

    run_sign(pa, pb, ps)
    run_sign(na, nb, ns)


def _sc_sims(zi, pa, pb, na, nb):
    return pl.kernel(
        _sc_body,
        out_type=[
            jax.ShapeDtypeStruct((E_PAD_,), jnp.float32),
            jax.ShapeDtypeStruct((E_PAD_,), jnp.float32),
        ],
        mesh=plsc.VectorSubcoreMesh(
            core_axis_name="c", subcore_axis_name="s"
        ),
        compiler_params=pltpu.CompilerParams(
            needs_layout_passes=False, use_tc_tiling_on_sc=False
        ),
        scratch_types=[
            pltpu.VMEM((EW_,), jnp.int32),
            pltpu.VMEM((EW_,), jnp.int32),
            pltpu.VMEM((K_, D_), jnp.float8_e4m3fn),
            pltpu.VMEM((K_, D_), jnp.float8_e4m3fn),
            pltpu.VMEM((K_, D_), jnp.float8_e4m3fn),
            pltpu.VMEM((K_, D_), jnp.float8_e4m3fn),
            pltpu.VMEM((EW_,), jnp.float32),
            pltpu.VMEM_SHARED((NROWS_, D_), jnp.float8_e4m3fn),
            pltpu.SemaphoreType.DMA,
            pltpu.SemaphoreType.DMA,
        ],
    )(zi, pa, pb, na, nb)


def _loss_body(ps_ref, ns_ref, out_ref, *, n_valid):
    rows, cols = ps_ref.shape
    ridx = lax.broadcasted_iota(jnp.int32, (rows, cols), 0)
    cidx = lax.broadcasted_iota(jnp.int32, (rows, cols), 1)
    valid = (ridx * cols + cidx) < n_valid
    ns = ns_ref[...]
    nsum = jnp.sum(jnp.where(valid, jnp.exp(ns), 0.0))
    ps = ps_ref[...]
    loss = jnp.where(valid, jnp.log(jnp.exp(ps) + nsum) - ps, 0.0)
    out_ref[...] = (jnp.sum(loss) / n_valid).reshape(1, 1)


def _loss(ps2d, ns2d, n_valid):
    out = pl.pallas_call(
        functools.partial(_loss_body, n_valid=n_valid),
        out_shape=jax.ShapeDtypeStruct((1, 1), jnp.float32),
    )(ps2d, ns2d)
    return out.reshape(())


def _pad_idx(v):
    return jnp.concatenate(
        [v, jnp.full((E_PAD_ - E_,), N_NODES_, jnp.int32)]
    )


def kernel(z, edge_index, negative_edge_index):
    zn = _normalize(z)
    zi = jnp.concatenate(
        [zn, jnp.zeros((NROWS_ - N_NODES_, D_), jnp.float8_e4m3fn)], axis=0
    )
    pa = _pad_idx(edge_index[0])
    pb = _pad_idx(edge_index[1])
    na = _pad_idx(negative_edge_index[0])
    nb = _pad_idx(negative_edge_index[1])
    ps, ns = _sc_sims(zi, pa, pb, na, nb)
    return _loss(
        ps.reshape(E_PAD_ // 128, 128), ns.reshape(E_PAD_ // 128, 128), E_
    )


# speedup vs baseline: 15.5757x; 1.0004x over previous
import functools

import jax
import jax.numpy as jnp
from jax import lax
from jax.experimental import pallas as pl
from jax.experimental.pallas import tpu as pltpu
from jax.experimental.pallas import tpu_sc as plsc

N_NODES_ = 10000
D_ = 256
E_ = 160000
INV_T = 10.0

NC_ = 2
NS_ = 16
NW_ = NC_ * NS_
K_ = 128
CW_ = 40
EW_ = K_ * CW_
E_PAD_ = NW_ * EW_
NROWS_ = 10016


def _norm_body(z_ref, zn_ref):
    x = z_ref[...]
    n2 = jnp.sum(x * x, axis=1, keepdims=True)
    inv = lax.rsqrt(jnp.maximum(n2, 1e-16))
    zn_ref[...] = (x * inv).astype(jnp.float8_e4m3fn)


def _normalize(z):
    return pl.pallas_call(
        _norm_body,
        out_shape=jax.ShapeDtypeStruct((N_NODES_, D_), jnp.float8_e4m3fn),
    )(z)


def _sc_body(
    zi, pa, pb, na, nb, ps, ns,
    idxa, idxb, ra0, rb0, ra1, rb1, sims, zs, sem0, sem1,
):
    sid = lax.axis_index("s")
    wid = sid * NC_ + lax.axis_index("c")
    iota16 = lax.iota(jnp.int32, 16)
    slots = ((ra0, rb0, sem0), (ra1, rb1, sem1))

    rows_per_tile = NROWS_ // NS_
    roff = pl.multiple_of(sid * rows_per_tile, rows_per_tile)
    pltpu.sync_copy(zi.at[pl.ds(roff, rows_per_tile)],
                    zs.at[pl.ds(roff, rows_per_tile)])
    plsc.subcore_barrier()

    def run_sign(ia_hbm, ib_hbm, out_hbm):
        base = pl.multiple_of(wid * EW_, EW_)
        pltpu.sync_copy(ia_hbm.at[pl.ds(base, EW_)], idxa)
        pltpu.sync_copy(ib_hbm.at[pl.ds(base, EW_)], idxb)

        def start(ci, slot):
            ra, rb, sem = slot
            lb = pl.multiple_of(ci * K_, K_)
            pltpu.async_copy(zs.at[idxa.at[pl.ds(lb, K_)]], ra, sem)
            pltpu.async_copy(zs.at[idxb.at[pl.ds(lb, K_)]], rb, sem)

        def wait(slot):
            ra, rb, sem = slot
            pltpu.make_async_copy(zi.at[pl.ds(0, K_)], ra, sem).wait()
            pltpu.make_async_copy(zi.at[pl.ds(0, K_)], rb, sem).wait()

        def compute(ci, slot):
            ra, rb, _ = slot
            lb = pl.multiple_of(ci * K_, K_)

            def gbody(g, carry):
                sims16 = jnp.zeros((16,), jnp.float32)
                for i in range(16):
                    e = g * 16 + i
                    acc0 = acc1 = None
                    for t in range(4):
                        a8 = ra[e, pl.ds(t * 64, 64)]
                        b8 = rb[e, pl.ds(t * 64, 64)]
                        alo, ahi = plsc.unpack(
                            a8,
                            format=plsc.PackFormat.INTERLEAVED,
                            preferred_element_type=jnp.bfloat16,
                        )
                        blo, bhi = plsc.unpack(
                            b8,
                            format=plsc.PackFormat.INTERLEAVED,
                            preferred_element_type=jnp.bfloat16,
                        )
                        p0 = alo * blo
                        p1 = ahi * bhi
                        acc0 = p0 if acc0 is None else acc0 + p0
                        acc1 = p1 if acc1 is None else acc1 + p1
                    lo, hi = plsc.unpack(
                        acc0 + acc1, format=plsc.PackFormat.INTERLEAVED
                    )
                    d = jnp.sum(lo + hi)
                    sims16 = jnp.where(iota16 == i, d, sims16)
                sims[ci, pl.ds(pl.multiple_of(g * 16, 16), 16)] = (
                    sims16 * INV_T
                )
                return carry

            lax.fori_loop(0, K_ // 16, gbody, 0)

        start(0, slots[0])

        def pair(p2, carry):
            for b2 in range(2):
                ci = p2 * 2 + b2
                nxt = ci + 1

                @pl.when(nxt < CW_)
                def _():
                    start(nxt, slots[1 - b2])

                wait(slots[b2])
                compute(ci, slots[b2])
            return carry

        lax.fori_loop(0, CW_ // 2, pair, 0)
        rbase = pl.multiple_of(wid * CW_, CW_)
        pltpu.sync_copy(sims, out_hbm.at[pl.ds(rbase, CW_)])

    run_sign(pa, pb, ps)
    run_sign(na, nb, ns)


def _sc_sims(zi, pa, pb, na, nb):
    return pl.kernel(
        _sc_body,
        out_type=[
            jax.ShapeDtypeStruct((E_PAD_ // K_, K_), jnp.float32),
            jax.ShapeDtypeStruct((E_PAD_ // K_, K_), jnp.float32),
        ],
        mesh=plsc.VectorSubcoreMesh(
            core_axis_name="c", subcore_axis_name="s"
        ),
        compiler_params=pltpu.CompilerParams(
            needs_layout_passes=False, use_tc_tiling_on_sc=False
        ),
        scratch_types=[
            pltpu.VMEM((EW_,), jnp.int32),
            pltpu.VMEM((EW_,), jnp.int32),
            pltpu.VMEM((K_, D_), jnp.float8_e4m3fn),
            pltpu.VMEM((K_, D_), jnp.float8_e4m3fn),
            pltpu.VMEM((K_, D_), jnp.float8_e4m3fn),
            pltpu.VMEM((K_, D_), jnp.float8_e4m3fn),
            pltpu.VMEM((CW_, K_), jnp.float32),
            pltpu.VMEM_SHARED((NROWS_, D_), jnp.float8_e4m3fn),
            pltpu.SemaphoreType.DMA,
            pltpu.SemaphoreType.DMA,
        ],
    )(zi, pa, pb, na, nb)


def _loss_body(ps_ref, ns_ref, out_ref, *, n_valid):
    rows, cols = ps_ref.shape
    ridx = lax.broadcasted_iota(jnp.int32, (rows, cols), 0)
    cidx = lax.broadcasted_iota(jnp.int32, (rows, cols), 1)
    valid = (ridx * cols + cidx) < n_valid
    ns = ns_ref[...]
    nsum = jnp.sum(jnp.where(valid, jnp.exp(ns), 0.0))
    ps = ps_ref[...]
    loss = jnp.where(valid, jnp.log(jnp.exp(ps) + nsum) - ps, 0.0)
    out_ref[...] = (jnp.sum(loss) / n_valid).reshape(1, 1)


def _loss(ps2d, ns2d, n_valid):
    out = pl.pallas_call(
        functools.partial(_loss_body, n_valid=n_valid),
        out_shape=jax.ShapeDtypeStruct((1, 1), jnp.float32),
    )(ps2d, ns2d)
    return out.reshape(())


def _pad_idx(v):
    return jnp.concatenate(
        [v, jnp.full((E_PAD_ - E_,), N_NODES_, jnp.int32)]
    )


def kernel(z, edge_index, negative_edge_index):
    zn = _normalize(z)
    zi = jnp.concatenate(
        [zn, jnp.zeros((NROWS_ - N_NODES_, D_), jnp.float8_e4m3fn)], axis=0
    )
    pa = _pad_idx(edge_index[0])
    pb = _pad_idx(edge_index[1])
    na = _pad_idx(negative_edge_index[0])
    nb = _pad_idx(negative_edge_index[1])
    ps, ns = _sc_sims(zi, pa, pb, na, nb)
    return _loss(ps, ns, E_)
